# baseline (speedup 1.0000x reference)
import functools

import jax
import jax.numpy as jnp
from jax import lax
from jax.experimental import pallas as pl
from jax.experimental.pallas import tpu as pltpu
from jax.experimental.pallas import tpu_sc as plsc

_MASK_RATIO = 0.75

_NC = 2
_NS = 16
_NW = _NC * _NS

_CHUNK = 128


def _shuffle_constants(B, N):
    len_keep = int(N * (1 - _MASK_RATIO))
    noise = jax.random.uniform(jax.random.key(1), (B, N), dtype=jnp.float32)
    ids_shuffle = jnp.argsort(noise, axis=1)
    ids_restore = jnp.argsort(ids_shuffle, axis=1)
    ids_keep = ids_shuffle[:, :len_keep]
    mask = ids_restore >= len_keep
    flat_idx = (jnp.arange(B, dtype=jnp.int32)[:, None] * N
                + ids_keep.astype(jnp.int32)).reshape(-1)
    return flat_idx, mask, ids_restore.astype(jnp.int32), len_keep


def _gather_rows(x_flat, flat_idx, R, D):
    rows_per_w = R // _NW
    n_chunks = rows_per_w // _CHUNK
    mesh = plsc.VectorSubcoreMesh(
        core_axis_name="c", subcore_axis_name="s",
        num_cores=_NC, num_subcores=_NS)

    @functools.partial(
        pl.kernel,
        out_type=jax.ShapeDtypeStruct((R, D), jnp.float32),
        mesh=mesh,
        scratch_types=[
            pltpu.VMEM((_CHUNK,), jnp.int32),
            pltpu.VMEM((_CHUNK, D), jnp.float32),
            pltpu.SemaphoreType.DMA,
        ],
        compiler_params=pltpu.CompilerParams(use_tc_tiling_on_sc=False),
    )
    def k(x_hbm, idx_hbm, out_hbm, idx_v, rows_v, sem):
        wid = lax.axis_index("s") * _NC + lax.axis_index("c")
        base = wid * rows_per_w
        for c in range(n_chunks):
            off = base + c * _CHUNK
            pltpu.sync_copy(idx_hbm.at[pl.ds(off, _CHUNK)], idx_v)
            pltpu.async_copy(x_hbm.at[idx_v], rows_v, sem).wait()
            pltpu.sync_copy(rows_v, out_hbm.at[pl.ds(off, _CHUNK)])

    return k(x_flat, flat_idx)


def kernel(x):
    B, N, D = x.shape
    flat_idx, mask, ids_restore, len_keep = _shuffle_constants(B, N)
    R = B * len_keep
    x_flat = x.reshape(B * N, D)
    x_masked = _gather_rows(x_flat, flat_idx, R, D).reshape(B, len_keep, D)
    return (x_masked, mask, ids_restore)


# speedup vs baseline: 1.0278x; 1.0278x over previous
import functools

import jax
import jax.numpy as jnp
from jax import lax
from jax.experimental import pallas as pl
from jax.experimental.pallas import tpu as pltpu
from jax.experimental.pallas import tpu_sc as plsc

_MASK_RATIO = 0.75

_NC = 2
_NS = 16
_NW = _NC * _NS

_CHUNK = 128


def _shuffle_constants(B, N):
    len_keep = int(N * (1 - _MASK_RATIO))
    noise = jax.random.uniform(jax.random.key(1), (B, N), dtype=jnp.float32)
    ids_shuffle = jnp.argsort(noise, axis=1)
    ids_restore = jnp.argsort(ids_shuffle, axis=1)
    ids_keep = ids_shuffle[:, :len_keep]
    mask = ids_restore >= len_keep
    flat_idx = (jnp.arange(B, dtype=jnp.int32)[:, None] * N
                + ids_keep.astype(jnp.int32)).reshape(-1)
    return flat_idx, mask, ids_restore.astype(jnp.int32), len_keep


_NBUF = 4


def _gather_rows(x_flat, flat_idx2d, R, D):
    rows_per_w = R // _NW
    n_chunks = rows_per_w // _CHUNK
    mesh = plsc.VectorSubcoreMesh(
        core_axis_name="c", subcore_axis_name="s",
        num_cores=_NC, num_subcores=_NS)

    @functools.partial(
        pl.kernel,
        out_type=jax.ShapeDtypeStruct((R, D), jnp.float32),
        mesh=mesh,
        scratch_types=[
            pltpu.VMEM((n_chunks, _CHUNK), jnp.int32),
            pltpu.VMEM((_NBUF, _CHUNK, D), jnp.float32),
            [pltpu.SemaphoreType.DMA] * _NBUF,
            [pltpu.SemaphoreType.DMA] * _NBUF,
        ],
        compiler_params=pltpu.CompilerParams(use_tc_tiling_on_sc=False),
    )
    def k(x_hbm, idx_hbm, out_hbm, idx_v, rows_v, gsem, ssem):
        wid = lax.axis_index("s") * _NC + lax.axis_index("c")
        base = wid * rows_per_w
        pltpu.sync_copy(idx_hbm.at[pl.ds(wid * n_chunks, n_chunks)], idx_v)
        gh = [None] * n_chunks
        sh = [None] * n_chunks
        for c in range(n_chunks + _NBUF - 1):
            if c < n_chunks:
                b = c % _NBUF
                if c >= _NBUF:
                    sh[c - _NBUF].wait()
                gh[c] = pltpu.async_copy(
                    x_hbm.at[idx_v.at[c]], rows_v.at[b], gsem[b])
            d = c - (_NBUF - 1)
            if 0 <= d < n_chunks:
                gh[d].wait()
                sh[d] = pltpu.async_copy(
                    rows_v.at[d % _NBUF],
                    out_hbm.at[pl.ds(base + d * _CHUNK, _CHUNK)],
                    ssem[d % _NBUF])
        for d in range(max(n_chunks - _NBUF, 0), n_chunks):
            sh[d].wait()

    return k(x_flat, flat_idx2d)


def kernel(x):
    B, N, D = x.shape
    flat_idx, mask, ids_restore, len_keep = _shuffle_constants(B, N)
    R = B * len_keep
    x_flat = x.reshape(B * N, D)
    flat_idx2d = flat_idx.reshape(R // _CHUNK, _CHUNK)
    x_masked = _gather_rows(x_flat, flat_idx2d, R, D).reshape(B, len_keep, D)
    return (x_masked, mask, ids_restore)


# speedup vs baseline: 1.9766x; 1.9232x over previous
import functools

import jax
import jax.numpy as jnp
from jax import lax
from jax.experimental import pallas as pl
from jax.experimental.pallas import tpu as pltpu
from jax.experimental.pallas import tpu_sc as plsc

_MASK_RATIO = 0.75

_NC = 2
_NS = 16
_NW = _NC * _NS

_C = 32
_L = 16


def _shuffle_constants(B, N):
    len_keep = int(N * (1 - _MASK_RATIO))
    noise = jax.random.uniform(jax.random.key(1), (B, N), dtype=jnp.float32)
    ids_shuffle = jnp.argsort(noise, axis=1)
    ids_restore = jnp.argsort(ids_shuffle, axis=1)
    ids_keep = ids_shuffle[:, :len_keep].astype(jnp.int32)
    mask = ids_restore >= len_keep
    return ids_keep, mask, ids_restore.astype(jnp.int32), len_keep


def _lane_gather(xt_flat, ids_keep, B, N, D, K):
    bpw = B // _NW
    nch = D // _C
    ngr = K // _L
    mesh = plsc.VectorSubcoreMesh(
        core_axis_name="c", subcore_axis_name="s",
        num_cores=_NC, num_subcores=_NS)

    @functools.partial(
        pl.kernel,
        out_type=jax.ShapeDtypeStruct((B * D, K), jnp.float32),
        mesh=mesh,
        scratch_types=[
            pltpu.VMEM((_C, N), jnp.float32),
            pltpu.VMEM((_C, K), jnp.float32),
            pltpu.VMEM((K,), jnp.int32),
            pltpu.SemaphoreType.DMA,
        ],
        compiler_params=pltpu.CompilerParams(needs_layout_passes=False),
    )
    def k(x_hbm, idx_hbm, out_hbm, inb, outb, idxv, sem):
        wid = lax.axis_index("s") * _NC + lax.axis_index("c")
        for bi in range(bpw):
            b = wid * bpw + bi
            pltpu.sync_copy(idx_hbm.at[b], idxv)
            cols = [idxv[pl.ds(_L * g, _L)] for g in range(ngr)]
            for c in range(nch):
                row0 = b * D + c * _C
                pltpu.async_copy(
                    x_hbm.at[pl.ds(row0, _C)], inb, sem).wait()

                def body(r, carry):
                    rr = jnp.full((_L,), r, dtype=jnp.int32)
                    for g in range(ngr):
                        v = plsc.load_gather(inb, [rr, cols[g]])
                        oc = lax.iota(jnp.int32, _L) + (_L * g)
                        plsc.store_scatter(outb, [rr, oc], v)
                    return carry

                lax.fori_loop(0, _C, body, 0)
                pltpu.async_copy(
                    outb, out_hbm.at[pl.ds(row0, _C)], sem).wait()

    return k(xt_flat, ids_keep)


def kernel(x):
    B, N, D = x.shape
    ids_keep, mask, ids_restore, len_keep = _shuffle_constants(B, N)
    xt_flat = x.transpose(0, 2, 1).reshape(B * D, N)
    out_t = _lane_gather(xt_flat, ids_keep, B, N, D, len_keep)
    x_masked = out_t.reshape(B, D, len_keep).transpose(0, 2, 1)
    return (x_masked, mask, ids_restore)


# speedup vs baseline: 2.9318x; 1.4832x over previous
import functools

import jax
import jax.numpy as jnp
from jax import lax
from jax.experimental import pallas as pl
from jax.experimental.pallas import tpu as pltpu
from jax.experimental.pallas import tpu_sc as plsc

_MASK_RATIO = 0.75

_NC = 2
_NS = 16
_NW = _NC * _NS

_C = 48
_L = 16


def _shuffle_constants(B, N):
    len_keep = int(N * (1 - _MASK_RATIO))
    noise = jax.random.uniform(jax.random.key(1), (B, N), dtype=jnp.float32)
    ids_shuffle = jnp.argsort(noise, axis=1)
    ids_restore = jnp.argsort(ids_shuffle, axis=1)
    ids_keep = ids_shuffle[:, :len_keep].astype(jnp.int32)
    mask = ids_restore >= len_keep
    return ids_keep, mask, ids_restore.astype(jnp.int32), len_keep


def _lane_gather(xt_flat, ids_keep, B, N, D, K):
    bpw = B // _NW
    nch = D // _C
    ngr = K // _L
    mesh = plsc.VectorSubcoreMesh(
        core_axis_name="c", subcore_axis_name="s",
        num_cores=_NC, num_subcores=_NS)

    @functools.partial(
        pl.kernel,
        out_type=jax.ShapeDtypeStruct((B * D, K), jnp.float32),
        mesh=mesh,
        scratch_types=[
            pltpu.VMEM((2, _C, N), jnp.float32),
            pltpu.VMEM((2, _C, K), jnp.float32),
            pltpu.VMEM((K,), jnp.int32),
            [pltpu.SemaphoreType.DMA] * 2,
            [pltpu.SemaphoreType.DMA] * 2,
        ],
        compiler_params=pltpu.CompilerParams(needs_layout_passes=False),
    )
    def k(x_hbm, idx_hbm, out_hbm, inb, outb, idxv, gsem, ssem):
        wid = lax.axis_index("s") * _NC + lax.axis_index("c")
        base = wid * bpw * D
        n_it = bpw * nch

        def start_in(i):
            bi, c = divmod(i, nch)
            s = i % 2
            return pltpu.async_copy(
                x_hbm.at[pl.ds(base + bi * D + c * _C, _C)],
                inb.at[s], gsem[s])

        in_h = [None] * n_it
        out_h = [None, None]
        cols = None
        in_h[0] = start_in(0)
        for i in range(n_it):
            bi, c = divmod(i, nch)
            s = i % 2
            if c == 0:
                pltpu.sync_copy(idx_hbm.at[wid * bpw + bi], idxv)
                cols = [idxv[pl.ds(_L * g, _L)] for g in range(ngr)]
            if i + 1 < n_it:
                in_h[i + 1] = start_in(i + 1)
            in_h[i].wait()
            if out_h[s] is not None:
                out_h[s].wait()

            def body(r, carry, _cols=cols, _s=s):
                rr = jnp.full((_L,), r, dtype=jnp.int32)
                for g in range(ngr):
                    v = plsc.load_gather(inb.at[_s], [rr, _cols[g]])
                    oc = lax.iota(jnp.int32, _L) + (_L * g)
                    plsc.store_scatter(outb.at[_s], [rr, oc], v)
                return carry

            lax.fori_loop(0, _C, body, 0)
            out_h[s] = pltpu.async_copy(
                outb.at[s],
                out_hbm.at[pl.ds(base + bi * D + c * _C, _C)], ssem[s])
        out_h[0].wait()
        out_h[1].wait()

    return k(xt_flat, ids_keep)


def kernel(x):
    B, N, D = x.shape
    ids_keep, mask, ids_restore, len_keep = _shuffle_constants(B, N)
    xt_flat = x.transpose(0, 2, 1).reshape(B * D, N)
    out_t = _lane_gather(xt_flat, ids_keep, B, N, D, len_keep)
    x_masked = out_t.reshape(B, D, len_keep).transpose(0, 2, 1)
    return (x_masked, mask, ids_restore)
